# baseline (speedup 1.0000x reference)
import functools

import jax
import jax.numpy as jnp
from jax import lax
from jax.experimental import pallas as pl
from jax.experimental.pallas import tpu as pltpu
from jax.experimental.pallas import tpu_sc as plsc

N_CODE = 1024
D_K = 256


_RB = 1024
_NCH = 128
_NCHUNKS = N_CODE // _NCH


def _dist_body(v_ref, c_ref, cn_ref, z_ref, lsum_ref):
    i = pl.program_id(0)

    @pl.when(i == 0)
    def _init():
        lsum_ref[0, 0] = 0.0

    v = v_ref[...]
    c = c_ref[...]
    s = lax.dot_general(v, c, (((1,), (1,)), ((), ())),
                        preferred_element_type=jnp.float32)
    vn = jnp.sum(v * v, axis=1, keepdims=True)
    cn = cn_ref[...]

    def chunk(j):
        lo, hi = j * _NCH, (j + 1) * _NCH
        return (vn + (-2.0) * s[:, lo:hi]) + cn[lo:hi][None, :]

    val = chunk(0)
    jwin = jnp.zeros((_RB, _NCH), jnp.int32)
    for j in range(1, _NCHUNKS):
        d = chunk(j)
        lt = d < val
        val = jnp.minimum(val, d)
        jwin = jnp.where(lt, j, jwin)
    g = jwin * _NCH + lax.broadcasted_iota(jnp.int32, (_RB, _NCH), 1)
    valT = val.T
    gT = g.T
    m = jnp.min(valT, axis=0, keepdims=True)
    z = jnp.min(jnp.where(valT == m, gT, N_CODE), axis=0)
    z_ref[...] = z.astype(jnp.int32)
    lsum_ref[0, 0] += jnp.sum(jnp.maximum(m, 0.0))


def _distances_argmin(v2, c, cn, part, n_part):
    steps = n_part // _RB
    grid = (steps,)
    z, lsum = pl.pallas_call(
        _dist_body,
        grid=grid,
        in_specs=[
            pl.BlockSpec((_RB, D_K), lambda i, p=part, s=steps: (p * s + i, 0)),
            pl.BlockSpec((N_CODE, D_K), lambda i: (0, 0)),
            pl.BlockSpec((N_CODE,), lambda i: (0,)),
        ],
        out_specs=[
            pl.BlockSpec((_RB,), lambda i: (i,)),
            pl.BlockSpec(memory_space=pltpu.SMEM),
        ],
        out_shape=[
            jax.ShapeDtypeStruct((n_part,), jnp.int32),
            jax.ShapeDtypeStruct((1, 1), jnp.float32),
        ],
    )(v2, c, cn)
    return z, lsum


_NC = 2
_NS = 16
_NW = _NC * _NS
_CHUNK = 32


def _sc_gather(table, idx, n_rows, out_rows, row0):
    b_per_w = n_rows // _NW
    n_chunk = b_per_w // _CHUNK
    mesh = plsc.VectorSubcoreMesh(core_axis_name="c", subcore_axis_name="s")

    @functools.partial(
        pl.kernel,
        mesh=mesh,
        out_type=jax.ShapeDtypeStruct((out_rows, D_K), jnp.float32),
        scratch_types=(
            [pltpu.VMEM((b_per_w,), jnp.int32)]
            + [pltpu.VMEM((_CHUNK, D_K), jnp.float32)] * n_chunk
            + [pltpu.SemaphoreType.DMA] * (2 * n_chunk)
        ),
    )
    def gather_k(table_hbm, idx_hbm, out_hbm, idx_v, *rest):
        bufs = rest[:n_chunk]
        gsem = rest[n_chunk:2 * n_chunk]
        wsem = rest[2 * n_chunk:]
        wid = lax.axis_index("s") * _NC + lax.axis_index("c")
        base = row0 + wid * b_per_w
        pltpu.sync_copy(idx_hbm.at[pl.ds(wid * b_per_w, b_per_w)], idx_v)
        g = [pltpu.async_copy(
                table_hbm.at[idx_v.at[pl.ds(j * _CHUNK, _CHUNK)]],
                bufs[j], gsem[j])
             for j in range(n_chunk)]
        w = []
        for j in range(n_chunk):
            g[j].wait()
            w.append(pltpu.async_copy(
                bufs[j], out_hbm.at[pl.ds(base + j * _CHUNK, _CHUNK)],
                wsem[j]))
        for cp in w:
            cp.wait()

    return gather_k(table, idx)


_NSPLIT = 4


def kernel(vecs, c_sum, c_count):
    b, r, cdim, k = vecs.shape
    n = b * r * cdim
    v2 = vecs.astype(jnp.float32).reshape(n, k)
    c = jnp.divide(c_sum, jnp.clip(jnp.expand_dims(c_count, -1), 0.01))
    c = c.astype(jnp.float32)
    cn = jnp.einsum('sk->s', jnp.square(c))

    np_ = n // _NSPLIT
    z_parts, lsums = [], []
    hat = None
    for p in range(_NSPLIT):
        z_p, lsum_p = _distances_argmin(v2, c, cn, p, np_)
        if p == 0:
            hat = _sc_gather(c, z_p, np_, n, 0)
        else:
            hat_p = _sc_gather(c, z_p, np_, np_, 0)
            hat = lax.dynamic_update_slice(hat, hat_p, (p * np_, 0))
        z_parts.append(z_p)
        lsums.append(lsum_p[0, 0])

    l_commit = sum(lsums) / (b * r)
    z = jnp.concatenate(z_parts).reshape(b, r, cdim)
    vecs_hat = hat.reshape(b, r, cdim, k).astype(vecs.dtype)
    l_codebook = jnp.zeros((), jnp.float32)
    return vecs_hat, z, l_commit, l_codebook


# speedup vs baseline: 1.1865x; 1.1865x over previous
import functools

import jax
import jax.numpy as jnp
from jax import lax
from jax.experimental import pallas as pl
from jax.experimental.pallas import tpu as pltpu
from jax.experimental.pallas import tpu_sc as plsc

N_CODE = 1024
D_K = 256


_RB = 1024
_NCH = 128
_NCHUNKS = N_CODE // _NCH


def _make_dist_body(with_gather):
    def body(*refs):
        if with_gather:
            v_ref, c_ref, cn_ref, _hat_ref, z_ref, lsum_ref, cz_ref = refs
        else:
            v_ref, c_ref, cn_ref, z_ref, lsum_ref = refs
        i = pl.program_id(0)

        @pl.when(i == 0)
        def _init():
            lsum_ref[0, 0] = 0.0

        v = v_ref[...]
        c = c_ref[...]
        s = lax.dot_general(v, c, (((1,), (1,)), ((), ())),
                            preferred_element_type=jnp.float32)
        vn = jnp.sum(v * v, axis=1, keepdims=True)
        cn = cn_ref[...]

        def chunk(j):
            lo, hi = j * _NCH, (j + 1) * _NCH
            return (vn + (-2.0) * s[:, lo:hi]) + cn[lo:hi][None, :]

        val = chunk(0)
        jwin = jnp.zeros((_RB, _NCH), jnp.int32)
        for j in range(1, _NCHUNKS):
            d = chunk(j)
            lt = d < val
            val = jnp.minimum(val, d)
            jwin = jnp.where(lt, j, jwin)
        g = jwin * _NCH + lax.broadcasted_iota(jnp.int32, (_RB, _NCH), 1)
        valT = val.T
        gT = g.T
        m = jnp.min(valT, axis=0, keepdims=True)
        z = jnp.min(jnp.where(valT == m, gT, N_CODE), axis=0)
        z_ref[...] = z.astype(jnp.int32)
        lsum_ref[0, 0] += jnp.sum(jnp.maximum(m, 0.0))
        if with_gather:
            zc = z[:, None]
            oh = jnp.where(
                lax.broadcasted_iota(jnp.int32, (_RB, N_CODE), 1) == zc,
                jnp.float32(1.0), jnp.float32(0.0))
            cz = lax.dot_general(oh, c, (((1,), (0,)), ((), ())),
                                 preferred_element_type=jnp.float32)
            cz_ref[...] = cz

    return body


def _distances_argmin(v2, c, cn, part, n_part):
    steps = n_part // _RB
    z, lsum = pl.pallas_call(
        _make_dist_body(False),
        grid=(steps,),
        in_specs=[
            pl.BlockSpec((_RB, D_K), lambda i, p=part, s=steps: (p * s + i, 0)),
            pl.BlockSpec((N_CODE, D_K), lambda i: (0, 0)),
            pl.BlockSpec((N_CODE,), lambda i: (0,)),
        ],
        out_specs=[
            pl.BlockSpec((_RB,), lambda i: (i,)),
            pl.BlockSpec(memory_space=pltpu.SMEM),
        ],
        out_shape=[
            jax.ShapeDtypeStruct((n_part,), jnp.int32),
            jax.ShapeDtypeStruct((1, 1), jnp.float32),
        ],
    )(v2, c, cn)
    return z, lsum


def _distances_argmin_gather(v2, c, cn, hat, part, n_part):
    n = v2.shape[0]
    steps = n_part // _RB
    z, lsum, hat = pl.pallas_call(
        _make_dist_body(True),
        grid=(steps,),
        in_specs=[
            pl.BlockSpec((_RB, D_K), lambda i, p=part, s=steps: (p * s + i, 0)),
            pl.BlockSpec((N_CODE, D_K), lambda i: (0, 0)),
            pl.BlockSpec((N_CODE,), lambda i: (0,)),
            pl.BlockSpec(memory_space=pl.ANY),
        ],
        out_specs=[
            pl.BlockSpec((_RB,), lambda i: (i,)),
            pl.BlockSpec(memory_space=pltpu.SMEM),
            pl.BlockSpec((_RB, D_K), lambda i, p=part, s=steps: (p * s + i, 0)),
        ],
        out_shape=[
            jax.ShapeDtypeStruct((n_part,), jnp.int32),
            jax.ShapeDtypeStruct((1, 1), jnp.float32),
            jax.ShapeDtypeStruct((n, D_K), jnp.float32),
        ],
        input_output_aliases={3: 2},
    )(v2, c, cn, hat)
    return z, lsum, hat


_NC = 2
_NS = 16
_NW = _NC * _NS
_CHUNK = 32


def _sc_gather(table, idx, n_rows, out_rows, row0):
    b_per_w = n_rows // _NW
    n_chunk = b_per_w // _CHUNK
    mesh = plsc.VectorSubcoreMesh(core_axis_name="c", subcore_axis_name="s")

    @functools.partial(
        pl.kernel,
        mesh=mesh,
        out_type=jax.ShapeDtypeStruct((out_rows, D_K), jnp.float32),
        scratch_types=(
            [pltpu.VMEM((b_per_w,), jnp.int32)]
            + [pltpu.VMEM((_CHUNK, D_K), jnp.float32)] * n_chunk
            + [pltpu.SemaphoreType.DMA] * (2 * n_chunk)
        ),
    )
    def gather_k(table_hbm, idx_hbm, out_hbm, idx_v, *rest):
        bufs = rest[:n_chunk]
        gsem = rest[n_chunk:2 * n_chunk]
        wsem = rest[2 * n_chunk:]
        wid = lax.axis_index("s") * _NC + lax.axis_index("c")
        base = row0 + wid * b_per_w
        pltpu.sync_copy(idx_hbm.at[pl.ds(wid * b_per_w, b_per_w)], idx_v)
        g = [pltpu.async_copy(
                table_hbm.at[idx_v.at[pl.ds(j * _CHUNK, _CHUNK)]],
                bufs[j], gsem[j])
             for j in range(n_chunk)]
        w = []
        for j in range(n_chunk):
            g[j].wait()
            w.append(pltpu.async_copy(
                bufs[j], out_hbm.at[pl.ds(base + j * _CHUNK, _CHUNK)],
                wsem[j]))
        for cp in w:
            cp.wait()

    return gather_k(table, idx)


_NSPLIT = 4


def kernel(vecs, c_sum, c_count):
    b, r, cdim, k = vecs.shape
    n = b * r * cdim
    v2 = vecs.astype(jnp.float32).reshape(n, k)
    c = jnp.divide(c_sum, jnp.clip(jnp.expand_dims(c_count, -1), 0.01))
    c = c.astype(jnp.float32)
    cn = jnp.einsum('sk->s', jnp.square(c))

    np_ = n // _NSPLIT
    z_parts, lsums = [], []
    z0, lsum0 = _distances_argmin(v2, c, cn, 0, np_)
    hat = _sc_gather(c, z0, np_, n, 0)
    z1, lsum1 = _distances_argmin(v2, c, cn, 1, np_)
    h1 = _sc_gather(c, z1, np_, np_, 0)
    z2, lsum2, hat = _distances_argmin_gather(v2, c, cn, hat, 2, np_)
    z3, lsum3, hat = _distances_argmin_gather(v2, c, cn, hat, 3, np_)
    hat = lax.dynamic_update_slice(hat, h1, (np_, 0))
    z_parts = [z0, z1, z2, z3]
    lsums = [lsum0[0, 0], lsum1[0, 0], lsum2[0, 0], lsum3[0, 0]]

    l_commit = sum(lsums) / (b * r)
    z = jnp.concatenate(z_parts).reshape(b, r, cdim)
    vecs_hat = hat.reshape(b, r, cdim, k).astype(vecs.dtype)
    l_codebook = jnp.zeros((), jnp.float32)
    return vecs_hat, z, l_commit, l_codebook


# speedup vs baseline: 1.2145x; 1.0236x over previous
import functools

import jax
import jax.numpy as jnp
from jax import lax
from jax.experimental import pallas as pl
from jax.experimental.pallas import tpu as pltpu
from jax.experimental.pallas import tpu_sc as plsc

N_CODE = 1024
D_K = 256


_RB = 2048
_NCH = 128
_NCHUNKS = N_CODE // _NCH


def _make_dist_body(with_gather):
    def body(*refs):
        if with_gather:
            v_ref, c_ref, cn_ref, _hat_ref, z_ref, lsum_ref, cz_ref = refs
        else:
            v_ref, c_ref, cn_ref, z_ref, lsum_ref = refs
        i = pl.program_id(0)

        @pl.when(i == 0)
        def _init():
            lsum_ref[0, 0] = 0.0

        v = v_ref[...]
        c = c_ref[...]
        s = lax.dot_general(v, c, (((1,), (1,)), ((), ())),
                            preferred_element_type=jnp.float32)
        vn = jnp.sum(v * v, axis=1, keepdims=True)
        cn = cn_ref[...]

        def chunk(j):
            lo, hi = j * _NCH, (j + 1) * _NCH
            return (vn + (-2.0) * s[:, lo:hi]) + cn[lo:hi][None, :]

        val = chunk(0)
        jwin = jnp.zeros((_RB, _NCH), jnp.int32)
        for j in range(1, _NCHUNKS):
            d = chunk(j)
            lt = d < val
            val = jnp.minimum(val, d)
            jwin = jnp.where(lt, j, jwin)
        g = jwin * _NCH + lax.broadcasted_iota(jnp.int32, (_RB, _NCH), 1)
        valT = val.T
        gT = g.T
        m = jnp.min(valT, axis=0, keepdims=True)
        z = jnp.min(jnp.where(valT == m, gT, N_CODE), axis=0)
        z_ref[...] = z.astype(jnp.int32)
        lsum_ref[0, 0] += jnp.sum(jnp.maximum(m, 0.0))
        if with_gather:
            zc = z[:, None]
            oh = jnp.where(
                lax.broadcasted_iota(jnp.int32, (_RB, N_CODE), 1) == zc,
                jnp.float32(1.0), jnp.float32(0.0))
            cz = lax.dot_general(oh, c, (((1,), (0,)), ((), ())),
                                 preferred_element_type=jnp.float32)
            cz_ref[...] = cz

    return body


def _distances_argmin(v2, c, cn, part, n_part):
    steps = n_part // _RB
    z, lsum = pl.pallas_call(
        _make_dist_body(False),
        grid=(steps,),
        in_specs=[
            pl.BlockSpec((_RB, D_K), lambda i, p=part, s=steps: (p * s + i, 0)),
            pl.BlockSpec((N_CODE, D_K), lambda i: (0, 0)),
            pl.BlockSpec((N_CODE,), lambda i: (0,)),
        ],
        out_specs=[
            pl.BlockSpec((_RB,), lambda i: (i,)),
            pl.BlockSpec(memory_space=pltpu.SMEM),
        ],
        out_shape=[
            jax.ShapeDtypeStruct((n_part,), jnp.int32),
            jax.ShapeDtypeStruct((1, 1), jnp.float32),
        ],
    )(v2, c, cn)
    return z, lsum


def _distances_argmin_gather(v2, c, cn, hat, part, n_part):
    n = v2.shape[0]
    steps = n_part // _RB
    z, lsum, hat = pl.pallas_call(
        _make_dist_body(True),
        grid=(steps,),
        in_specs=[
            pl.BlockSpec((_RB, D_K), lambda i, p=part, s=steps: (p * s + i, 0)),
            pl.BlockSpec((N_CODE, D_K), lambda i: (0, 0)),
            pl.BlockSpec((N_CODE,), lambda i: (0,)),
            pl.BlockSpec(memory_space=pl.ANY),
        ],
        out_specs=[
            pl.BlockSpec((_RB,), lambda i: (i,)),
            pl.BlockSpec(memory_space=pltpu.SMEM),
            pl.BlockSpec((_RB, D_K), lambda i, p=part, s=steps: (p * s + i, 0)),
        ],
        out_shape=[
            jax.ShapeDtypeStruct((n_part,), jnp.int32),
            jax.ShapeDtypeStruct((1, 1), jnp.float32),
            jax.ShapeDtypeStruct((n, D_K), jnp.float32),
        ],
        input_output_aliases={3: 2},
    )(v2, c, cn, hat)
    return z, lsum, hat


_NC = 2
_NS = 16
_NW = _NC * _NS
_CHUNK = 32


def _sc_gather(table, idx, n_rows, out_rows, row0):
    b_per_w = n_rows // _NW
    n_chunk = b_per_w // _CHUNK
    mesh = plsc.VectorSubcoreMesh(core_axis_name="c", subcore_axis_name="s")

    @functools.partial(
        pl.kernel,
        mesh=mesh,
        out_type=jax.ShapeDtypeStruct((out_rows, D_K), jnp.float32),
        scratch_types=(
            [pltpu.VMEM((b_per_w,), jnp.int32)]
            + [pltpu.VMEM((_CHUNK, D_K), jnp.float32)] * n_chunk
            + [pltpu.SemaphoreType.DMA] * (2 * n_chunk)
        ),
    )
    def gather_k(table_hbm, idx_hbm, out_hbm, idx_v, *rest):
        bufs = rest[:n_chunk]
        gsem = rest[n_chunk:2 * n_chunk]
        wsem = rest[2 * n_chunk:]
        wid = lax.axis_index("s") * _NC + lax.axis_index("c")
        base = row0 + wid * b_per_w
        pltpu.sync_copy(idx_hbm.at[pl.ds(wid * b_per_w, b_per_w)], idx_v)
        g = [pltpu.async_copy(
                table_hbm.at[idx_v.at[pl.ds(j * _CHUNK, _CHUNK)]],
                bufs[j], gsem[j])
             for j in range(n_chunk)]
        w = []
        for j in range(n_chunk):
            g[j].wait()
            w.append(pltpu.async_copy(
                bufs[j], out_hbm.at[pl.ds(base + j * _CHUNK, _CHUNK)],
                wsem[j]))
        for cp in w:
            cp.wait()

    return gather_k(table, idx)


_NSPLIT = 4


def kernel(vecs, c_sum, c_count):
    b, r, cdim, k = vecs.shape
    n = b * r * cdim
    v2 = vecs.astype(jnp.float32).reshape(n, k)
    c = jnp.divide(c_sum, jnp.clip(jnp.expand_dims(c_count, -1), 0.01))
    c = c.astype(jnp.float32)
    cn = jnp.einsum('sk->s', jnp.square(c))

    np_ = n // _NSPLIT
    z_parts, lsums = [], []
    z0, lsum0 = _distances_argmin(v2, c, cn, 0, np_)
    hat = _sc_gather(c, z0, np_, n, 0)
    z1, lsum1 = _distances_argmin(v2, c, cn, 1, np_)
    h1 = _sc_gather(c, z1, np_, np_, 0)
    z2, lsum2, hat = _distances_argmin_gather(v2, c, cn, hat, 2, np_)
    z3, lsum3, hat = _distances_argmin_gather(v2, c, cn, hat, 3, np_)
    hat = lax.dynamic_update_slice(hat, h1, (np_, 0))
    z_parts = [z0, z1, z2, z3]
    lsums = [lsum0[0, 0], lsum1[0, 0], lsum2[0, 0], lsum3[0, 0]]

    l_commit = sum(lsums) / (b * r)
    z = jnp.concatenate(z_parts).reshape(b, r, cdim)
    vecs_hat = hat.reshape(b, r, cdim, k).astype(vecs.dtype)
    l_codebook = jnp.zeros((), jnp.float32)
    return vecs_hat, z, l_commit, l_codebook
